# baseline (speedup 1.0000x reference)
import functools

import jax
import jax.numpy as jnp
from jax import lax
from jax.experimental import pallas as pl
from jax.experimental.pallas import tpu as pltpu
from jax.experimental.pallas import tpu_sc as plsc

N_NODES = 10000
N_EDGES = 320000
D = 128
DH = D // 2

NC = 2
NS = 16
NW = NC * NS
CHUNK = 80
NCH1 = N_EDGES // NW // CHUNK
CHUNK3 = 125
NCH3 = N_EDGES // NW // CHUNK3
NBUF = 4
N_PAD = 10240
ROWS_PER_TILE = N_PAD // NS

_mesh = plsc.VectorSubcoreMesh(
    core_axis_name="c", subcore_axis_name="s", num_cores=NC, num_subcores=NS)
_sc_params = pltpu.CompilerParams(use_tc_tiling_on_sc=False)


@functools.partial(
    pl.kernel,
    out_type=jax.ShapeDtypeStruct((NC, N_PAD), jnp.float32),
    mesh=_mesh,
    scratch_types=[
        pltpu.VMEM((NCH1, CHUNK), jnp.int32),
        pltpu.VMEM((CHUNK,), jnp.float32),
        pltpu.VMEM((ROWS_PER_TILE,), jnp.float32),
        pltpu.VMEM_SHARED((N_PAD,), jnp.float32),
    ],
    compiler_params=_sc_params,
)
def _deg_kernel(dst_hbm, out_hbm, idx_v, ones_v, zeros_v, deg_sh):
    c = lax.axis_index("c")
    s = lax.axis_index("s")
    wid = c * NS + s

    def fill(i, _):
        ones_v[pl.ds(i * 16, 16)] = jnp.full((16,), 1.0, jnp.float32)
        return 0
    lax.fori_loop(0, CHUNK // 16, fill, 0)

    def fillz(i, _):
        zeros_v[pl.ds(i * 16, 16)] = jnp.zeros((16,), jnp.float32)
        return 0
    lax.fori_loop(0, ROWS_PER_TILE // 16, fillz, 0)

    pltpu.sync_copy(zeros_v, deg_sh.at[pl.ds(s * ROWS_PER_TILE, ROWS_PER_TILE)])
    plsc.subcore_barrier()

    pltpu.sync_copy(dst_hbm.at[wid], idx_v)

    def body(g, _):
        pltpu.sync_copy(ones_v, deg_sh.at[idx_v.at[g]], add=True)
        return 0
    lax.fori_loop(0, NCH1, body, 0)

    plsc.subcore_barrier()
    pltpu.sync_copy(deg_sh.at[pl.ds(s * ROWS_PER_TILE, ROWS_PER_TILE)],
                    out_hbm.at[c, pl.ds(s * ROWS_PER_TILE, ROWS_PER_TILE)])


def _scale_body(degT_ref, x_ref, x2_ref):
    deg = degT_ref[:, 0:1] + degT_ref[:, 1:2] + 1.0
    dinv = lax.rsqrt(deg)
    x2_ref[...] = (x_ref[...] * dinv).astype(jnp.bfloat16)


def _scale_call(degT, x):
    return pl.pallas_call(
        _scale_body,
        out_shape=jax.ShapeDtypeStruct((N_NODES, D), jnp.bfloat16),
    )(degT, x)


@functools.partial(
    pl.kernel,
    out_type=jax.ShapeDtypeStruct((NC, N_PAD, D), jnp.bfloat16),
    mesh=_mesh,
    scratch_types=[
        pltpu.VMEM((NCH3, CHUNK3), jnp.int32),
        pltpu.VMEM((NCH3, CHUNK3), jnp.int32),
        pltpu.VMEM((NBUF, CHUNK3, D), jnp.bfloat16),
        pltpu.VMEM((128, D), jnp.bfloat16),
        pltpu.VMEM_SHARED((N_PAD, D), jnp.bfloat16),
        [pltpu.SemaphoreType.DMA] * NBUF,
        [pltpu.SemaphoreType.DMA] * NBUF,
    ],
    compiler_params=_sc_params,
)
def _agg_kernel(src_hbm, dst_hbm, x2_hbm, out_hbm,
                src_v, dst_v, rows_v, zeros_v, agg_sh, gsems, ssems):
    c = lax.axis_index("c")
    s = lax.axis_index("s")
    wid = c * NS + s

    def fillz(i, _):
        r = i // 4
        col = (i - r * 4) * 32
        zeros_v[r, pl.ds(col, 32)] = jnp.zeros((32,), jnp.bfloat16)
        return 0
    lax.fori_loop(0, 128 * (D // 32), fillz, 0)

    def zrow(k, _):
        pltpu.sync_copy(
            zeros_v, agg_sh.at[pl.ds(s * ROWS_PER_TILE + k * 128, 128)])
        return 0
    lax.fori_loop(0, ROWS_PER_TILE // 128, zrow, 0)
    plsc.subcore_barrier()

    pltpu.sync_copy(src_hbm.at[wid], src_v)
    pltpu.sync_copy(dst_hbm.at[wid], dst_v)

    def gather(g, b):
        pltpu.async_copy(x2_hbm.at[src_v.at[g]], rows_v.at[b], gsems[b])

    def gwait(g, b):
        pltpu.make_async_copy(
            x2_hbm.at[src_v.at[g]], rows_v.at[b], gsems[b]).wait()

    def scatter(g, b):
        pltpu.async_copy(rows_v.at[b], agg_sh.at[dst_v.at[g]], ssems[b],
                         add=True)

    def swait(g, b):
        pltpu.make_async_copy(rows_v.at[b], agg_sh.at[dst_v.at[g]],
                              ssems[b]).wait()

    for b in range(NBUF):
        gather(b, b)

    def body(p, _):
        g0 = p * NBUF
        for b in range(NBUF):
            gwait(g0 + b, b)
            scatter(g0 + b, b)
        for b in range(NBUF):
            swait(g0 + b, b)
            gather(g0 + NBUF + b, b)
        return 0
    lax.fori_loop(0, NCH3 // NBUF - 1, body, 0)

    g0 = NCH3 - NBUF
    for b in range(NBUF):
        gwait(g0 + b, b)
        scatter(g0 + b, b)
    for b in range(NBUF):
        swait(g0 + b, b)

    plsc.subcore_barrier()
    pltpu.sync_copy(
        agg_sh.at[pl.ds(s * ROWS_PER_TILE, ROWS_PER_TILE)],
        out_hbm.at[c, pl.ds(s * ROWS_PER_TILE, ROWS_PER_TILE)])


def _mlp_body(agg_ref, x_ref, degT_ref, wg_ref, bg_ref, w1_ref, b1_ref,
              w2_ref, b2_ref, out_ref):
    deg = degT_ref[:, 0:1] + degT_ref[:, 1:2] + 1.0
    dinv = lax.rsqrt(deg)
    S = (agg_ref[0, :N_NODES, :].astype(jnp.float32)
         + agg_ref[1, :N_NODES, :].astype(jnp.float32))
    y = dinv * (S + dinv * x_ref[...])
    gcn = jnp.dot(y, wg_ref[...], preferred_element_type=jnp.float32) + bg_ref[...]
    h1 = jnp.maximum(
        jnp.dot(gcn, w1_ref[...], preferred_element_type=jnp.float32) + b1_ref[...],
        0.0)
    out_ref[...] = (
        jnp.dot(h1, w2_ref[...], preferred_element_type=jnp.float32) + b2_ref[...])


def _mlp_call(agg, x, degT, W_gcn, b_gcn, W1, b1, W2, b2):
    return pl.pallas_call(
        _mlp_body,
        out_shape=jax.ShapeDtypeStruct((N_NODES, D), jnp.float32),
    )(agg, x, degT, W_gcn, b_gcn.reshape(1, D), W1, b1.reshape(1, D),
      W2, b2.reshape(1, D))


def kernel(x, edge_index, W_gcn, b_gcn, W1, b1, W2, b2):
    src = edge_index[0].astype(jnp.int32)
    dst = edge_index[1].astype(jnp.int32)
    dst1 = dst.reshape(NW, NCH1, CHUNK)
    src3 = src.reshape(NW, NCH3, CHUNK3)
    dst3 = dst.reshape(NW, NCH3, CHUNK3)

    deg_part = _deg_kernel(dst1)
    degT = deg_part[:, :N_NODES].T
    x2 = _scale_call(degT, x)
    agg = _agg_kernel(src3, dst3, x2)
    return _mlp_call(agg, x, degT, W_gcn, b_gcn, W1, b1, W2, b2)


# speedup vs baseline: 1.1052x; 1.1052x over previous
import functools

import jax
import jax.numpy as jnp
from jax import lax
from jax.experimental import pallas as pl
from jax.experimental.pallas import tpu as pltpu
from jax.experimental.pallas import tpu_sc as plsc

N_NODES = 10000
N_EDGES = 320000
D = 128
DH = D // 2

NC = 2
NS = 16
NW = NC * NS
CHUNK = 80
NCH1 = N_EDGES // NW // CHUNK
CHUNK3 = 125
NCH3 = N_EDGES // NS // CHUNK3
NBUF = 8
N_PAD = 10240
ROWS_PER_TILE = N_PAD // NS

_mesh = plsc.VectorSubcoreMesh(
    core_axis_name="c", subcore_axis_name="s", num_cores=NC, num_subcores=NS)
_sc_params = pltpu.CompilerParams(use_tc_tiling_on_sc=False)


@functools.partial(
    pl.kernel,
    out_type=jax.ShapeDtypeStruct((NC, N_PAD), jnp.float32),
    mesh=_mesh,
    scratch_types=[
        pltpu.VMEM((NCH1, CHUNK), jnp.int32),
        pltpu.VMEM((CHUNK,), jnp.float32),
        pltpu.VMEM((ROWS_PER_TILE,), jnp.float32),
        pltpu.VMEM_SHARED((N_PAD,), jnp.float32),
    ],
    compiler_params=_sc_params,
)
def _deg_kernel(dst_hbm, out_hbm, idx_v, ones_v, zeros_v, deg_sh):
    c = lax.axis_index("c")
    s = lax.axis_index("s")
    wid = c * NS + s

    def fill(i, _):
        ones_v[pl.ds(i * 16, 16)] = jnp.full((16,), 1.0, jnp.float32)
        return 0
    lax.fori_loop(0, CHUNK // 16, fill, 0)

    def fillz(i, _):
        zeros_v[pl.ds(i * 16, 16)] = jnp.zeros((16,), jnp.float32)
        return 0
    lax.fori_loop(0, ROWS_PER_TILE // 16, fillz, 0)

    pltpu.sync_copy(zeros_v, deg_sh.at[pl.ds(s * ROWS_PER_TILE, ROWS_PER_TILE)])
    plsc.subcore_barrier()

    pltpu.sync_copy(dst_hbm.at[wid], idx_v)

    def body(g, _):
        pltpu.sync_copy(ones_v, deg_sh.at[idx_v.at[g]], add=True)
        return 0
    lax.fori_loop(0, NCH1, body, 0)

    plsc.subcore_barrier()
    pltpu.sync_copy(deg_sh.at[pl.ds(s * ROWS_PER_TILE, ROWS_PER_TILE)],
                    out_hbm.at[c, pl.ds(s * ROWS_PER_TILE, ROWS_PER_TILE)])


def _scale_body(degT_ref, x_ref, x2_ref):
    deg = degT_ref[:, 0:1] + degT_ref[:, 1:2] + 1.0
    dinv = lax.rsqrt(deg)
    x2_ref[...] = (x_ref[...] * dinv).astype(jnp.bfloat16)


def _scale_call(degT, x):
    return pl.pallas_call(
        _scale_body,
        out_shape=jax.ShapeDtypeStruct((N_NODES, D), jnp.bfloat16),
    )(degT, x)


@functools.partial(
    pl.kernel,
    out_type=jax.ShapeDtypeStruct((N_PAD, D), jnp.bfloat16),
    mesh=_mesh,
    scratch_types=[
        pltpu.VMEM((NCH3, CHUNK3), jnp.int32),
        pltpu.VMEM((NCH3, CHUNK3), jnp.int32),
        pltpu.VMEM((NBUF, CHUNK3, DH), jnp.bfloat16),
        pltpu.VMEM((128, DH), jnp.bfloat16),
        pltpu.VMEM_SHARED((N_PAD, DH), jnp.bfloat16),
        [pltpu.SemaphoreType.DMA] * NBUF,
        [pltpu.SemaphoreType.DMA] * NBUF,
    ],
    compiler_params=_sc_params,
)
def _agg_kernel(src_hbm, dst_hbm, x2_hbm, out_hbm,
                src_v, dst_v, rows_v, zeros_v, agg_sh, gsems, ssems):
    c = lax.axis_index("c")
    s = lax.axis_index("s")

    def fillz(i, _):
        r = i // 2
        col = (i - r * 2) * 32
        zeros_v[r, pl.ds(col, 32)] = jnp.zeros((32,), jnp.bfloat16)
        return 0
    lax.fori_loop(0, 128 * (DH // 32), fillz, 0)

    def zrow(k, _):
        pltpu.sync_copy(
            zeros_v, agg_sh.at[pl.ds(s * ROWS_PER_TILE + k * 128, 128)])
        return 0
    lax.fori_loop(0, ROWS_PER_TILE // 128, zrow, 0)
    plsc.subcore_barrier()

    pltpu.sync_copy(src_hbm.at[c, s], src_v)
    pltpu.sync_copy(dst_hbm.at[s], dst_v)

    def gather(g, b):
        pltpu.async_copy(x2_hbm.at[src_v.at[g]], rows_v.at[b], gsems[b])

    def gwait(g, b):
        pltpu.make_async_copy(
            x2_hbm.at[src_v.at[g]], rows_v.at[b], gsems[b]).wait()

    def scatter(g, b):
        pltpu.async_copy(rows_v.at[b], agg_sh.at[dst_v.at[g]], ssems[b],
                         add=True)

    def swait(g, b):
        pltpu.make_async_copy(rows_v.at[b], agg_sh.at[dst_v.at[g]],
                              ssems[b]).wait()

    for b in range(NBUF):
        gather(b, b)

    def body(p, _):
        g0 = p * NBUF
        for b in range(NBUF):
            gwait(g0 + b, b)
            scatter(g0 + b, b)
        for b in range(NBUF):
            swait(g0 + b, b)
            gather(g0 + NBUF + b, b)
        return 0
    lax.fori_loop(0, NCH3 // NBUF - 1, body, 0)

    g0 = NCH3 - NBUF
    for b in range(NBUF):
        gwait(g0 + b, b)
        scatter(g0 + b, b)
    for b in range(NBUF):
        swait(g0 + b, b)

    plsc.subcore_barrier()
    pltpu.sync_copy(
        agg_sh.at[pl.ds(s * ROWS_PER_TILE, ROWS_PER_TILE)],
        out_hbm.at[pl.ds(s * ROWS_PER_TILE, ROWS_PER_TILE), pl.ds(c * DH, DH)])


def _mlp_body(agg_ref, x_ref, degT_ref, wg_ref, bg_ref, w1_ref, b1_ref,
              w2_ref, b2_ref, out_ref):
    deg = degT_ref[:, 0:1] + degT_ref[:, 1:2] + 1.0
    dinv = lax.rsqrt(deg)
    y = dinv * (agg_ref[:N_NODES, :].astype(jnp.float32) + dinv * x_ref[...])
    gcn = jnp.dot(y, wg_ref[...], preferred_element_type=jnp.float32) + bg_ref[...]
    h1 = jnp.maximum(
        jnp.dot(gcn, w1_ref[...], preferred_element_type=jnp.float32) + b1_ref[...],
        0.0)
    out_ref[...] = (
        jnp.dot(h1, w2_ref[...], preferred_element_type=jnp.float32) + b2_ref[...])


def _mlp_call(agg, x, degT, W_gcn, b_gcn, W1, b1, W2, b2):
    return pl.pallas_call(
        _mlp_body,
        out_shape=jax.ShapeDtypeStruct((N_NODES, D), jnp.float32),
    )(agg, x, degT, W_gcn, b_gcn.reshape(1, D), W1, b1.reshape(1, D),
      W2, b2.reshape(1, D))


def kernel(x, edge_index, W_gcn, b_gcn, W1, b1, W2, b2):
    src = edge_index[0].astype(jnp.int32)
    dst = edge_index[1].astype(jnp.int32)
    dst1 = dst.reshape(NW, NCH1, CHUNK)
    src2 = src * 2
    src3 = jnp.stack([src2, src2 + 1]).reshape(NC, NS, NCH3, CHUNK3)
    dst3 = dst.reshape(NS, NCH3, CHUNK3)

    deg_part = _deg_kernel(dst1)
    degT = deg_part[:, :N_NODES].T
    x2 = _scale_call(degT, x)
    x2v = x2.reshape(2 * N_NODES, DH)
    agg = _agg_kernel(src3, dst3, x2v)
    return _mlp_call(agg, x, degT, W_gcn, b_gcn, W1, b1, W2, b2)


# speedup vs baseline: 1.1220x; 1.0153x over previous
import functools

import jax
import jax.numpy as jnp
from jax import lax
from jax.experimental import pallas as pl
from jax.experimental.pallas import tpu as pltpu
from jax.experimental.pallas import tpu_sc as plsc

N_NODES = 10000
N_EDGES = 320000
D = 128
DH = D // 2

NC = 2
NS = 16
NW = NC * NS
CHUNK = 80
NCH1 = N_EDGES // NW // CHUNK
CHUNK3 = 125
NCH3 = N_EDGES // NS // CHUNK3
NBUF = 8
N_PAD = 10240
ROWS_PER_TILE = N_PAD // NS

_mesh = plsc.VectorSubcoreMesh(
    core_axis_name="c", subcore_axis_name="s", num_cores=NC, num_subcores=NS)
_sc_params = pltpu.CompilerParams(use_tc_tiling_on_sc=False)


@functools.partial(
    pl.kernel,
    out_type=jax.ShapeDtypeStruct((NC, N_PAD), jnp.float32),
    mesh=_mesh,
    scratch_types=[
        pltpu.VMEM((NCH1, CHUNK), jnp.int32),
        pltpu.VMEM((CHUNK,), jnp.float32),
        pltpu.VMEM((ROWS_PER_TILE,), jnp.float32),
        pltpu.VMEM_SHARED((N_PAD,), jnp.float32),
        pltpu.SemaphoreType.DMA,
    ],
    compiler_params=_sc_params,
)
def _deg_kernel(dst_hbm, out_hbm, idx_v, ones_v, zeros_v, deg_sh, ssem):
    c = lax.axis_index("c")
    s = lax.axis_index("s")
    wid = c * NS + s

    def fill(i, _):
        ones_v[pl.ds(i * 16, 16)] = jnp.full((16,), 1.0, jnp.float32)
        return 0
    lax.fori_loop(0, CHUNK // 16, fill, 0)

    def fillz(i, _):
        zeros_v[pl.ds(i * 16, 16)] = jnp.zeros((16,), jnp.float32)
        return 0
    lax.fori_loop(0, ROWS_PER_TILE // 16, fillz, 0)

    pltpu.sync_copy(zeros_v, deg_sh.at[pl.ds(s * ROWS_PER_TILE, ROWS_PER_TILE)])
    plsc.subcore_barrier()

    pltpu.sync_copy(dst_hbm.at[wid], idx_v)

    def prol(g, _):
        pltpu.async_copy(ones_v, deg_sh.at[idx_v.at[g]], ssem, add=True)
        return 0
    lax.fori_loop(0, 8, prol, 0)

    def body(g, _):
        pltpu.make_async_copy(ones_v, deg_sh.at[idx_v.at[g]], ssem).wait()
        pltpu.async_copy(ones_v, deg_sh.at[idx_v.at[g + 8]], ssem, add=True)
        return 0
    lax.fori_loop(0, NCH1 - 8, body, 0)

    def drain(g, _):
        pltpu.make_async_copy(ones_v, deg_sh.at[idx_v.at[g]], ssem).wait()
        return 0
    lax.fori_loop(NCH1 - 8, NCH1, drain, 0)

    plsc.subcore_barrier()
    pltpu.sync_copy(deg_sh.at[pl.ds(s * ROWS_PER_TILE, ROWS_PER_TILE)],
                    out_hbm.at[c, pl.ds(s * ROWS_PER_TILE, ROWS_PER_TILE)])


def _scale_body(degT_ref, x_ref, x2_ref):
    deg = degT_ref[:, 0:1] + degT_ref[:, 1:2] + 1.0
    dinv = lax.rsqrt(deg)
    x2_ref[...] = (x_ref[...] * dinv).astype(jnp.bfloat16)


def _scale_call(degT, x):
    return pl.pallas_call(
        _scale_body,
        out_shape=jax.ShapeDtypeStruct((N_NODES, D), jnp.bfloat16),
    )(degT, x)


@functools.partial(
    pl.kernel,
    out_type=jax.ShapeDtypeStruct((N_PAD, D), jnp.bfloat16),
    mesh=_mesh,
    scratch_types=[
        pltpu.VMEM((NCH3, CHUNK3), jnp.int32),
        pltpu.VMEM((NCH3, CHUNK3), jnp.int32),
        pltpu.VMEM((NBUF, CHUNK3, DH), jnp.bfloat16),
        pltpu.VMEM((128, DH), jnp.bfloat16),
        pltpu.VMEM_SHARED((N_PAD, DH), jnp.bfloat16),
        [pltpu.SemaphoreType.DMA] * NBUF,
        [pltpu.SemaphoreType.DMA] * NBUF,
    ],
    compiler_params=_sc_params,
)
def _agg_kernel(src_hbm, dst_hbm, x2_hbm, out_hbm,
                src_v, dst_v, rows_v, zeros_v, agg_sh, gsems, ssems):
    c = lax.axis_index("c")
    s = lax.axis_index("s")

    def fillz(i, _):
        r = i // 2
        col = (i - r * 2) * 32
        zeros_v[r, pl.ds(col, 32)] = jnp.zeros((32,), jnp.bfloat16)
        return 0
    lax.fori_loop(0, 128 * (DH // 32), fillz, 0)

    def zrow(k, _):
        pltpu.sync_copy(
            zeros_v, agg_sh.at[pl.ds(s * ROWS_PER_TILE + k * 128, 128)])
        return 0
    lax.fori_loop(0, ROWS_PER_TILE // 128, zrow, 0)
    plsc.subcore_barrier()

    pltpu.sync_copy(src_hbm.at[c, s], src_v)
    pltpu.sync_copy(dst_hbm.at[s], dst_v)

    def gather(g, b):
        pltpu.async_copy(x2_hbm.at[src_v.at[g]], rows_v.at[b], gsems[b])

    def gwait(g, b):
        pltpu.make_async_copy(
            x2_hbm.at[src_v.at[g]], rows_v.at[b], gsems[b]).wait()

    def scatter(g, b):
        pltpu.async_copy(rows_v.at[b], agg_sh.at[dst_v.at[g]], ssems[b],
                         add=True)

    def swait(g, b):
        pltpu.make_async_copy(rows_v.at[b], agg_sh.at[dst_v.at[g]],
                              ssems[b]).wait()

    for b in range(NBUF):
        gather(b, b)

    def body(p, _):
        g0 = p * NBUF
        for b in range(NBUF):
            gwait(g0 + b, b)
            scatter(g0 + b, b)
        for b in range(NBUF):
            swait(g0 + b, b)
            gather(g0 + NBUF + b, b)
        return 0
    lax.fori_loop(0, NCH3 // NBUF - 1, body, 0)

    g0 = NCH3 - NBUF
    for b in range(NBUF):
        gwait(g0 + b, b)
        scatter(g0 + b, b)
    for b in range(NBUF):
        swait(g0 + b, b)

    plsc.subcore_barrier()
    pltpu.sync_copy(
        agg_sh.at[pl.ds(s * ROWS_PER_TILE, ROWS_PER_TILE)],
        out_hbm.at[pl.ds(s * ROWS_PER_TILE, ROWS_PER_TILE), pl.ds(c * DH, DH)])


def _mlp_body(agg_ref, x_ref, degT_ref, wg_ref, bg_ref, w1_ref, b1_ref,
              w2_ref, b2_ref, out_ref):
    deg = degT_ref[:, 0:1] + degT_ref[:, 1:2] + 1.0
    dinv = lax.rsqrt(deg)
    y = dinv * (agg_ref[:N_NODES, :].astype(jnp.float32) + dinv * x_ref[...])
    gcn = jnp.dot(y, wg_ref[...], preferred_element_type=jnp.float32) + bg_ref[...]
    h1 = jnp.maximum(
        jnp.dot(gcn, w1_ref[...], preferred_element_type=jnp.float32) + b1_ref[...],
        0.0)
    out_ref[...] = (
        jnp.dot(h1, w2_ref[...], preferred_element_type=jnp.float32) + b2_ref[...])


def _mlp_call(agg, x, degT, W_gcn, b_gcn, W1, b1, W2, b2):
    return pl.pallas_call(
        _mlp_body,
        out_shape=jax.ShapeDtypeStruct((N_NODES, D), jnp.float32),
    )(agg, x, degT, W_gcn, b_gcn.reshape(1, D), W1, b1.reshape(1, D),
      W2, b2.reshape(1, D))


def kernel(x, edge_index, W_gcn, b_gcn, W1, b1, W2, b2):
    src = edge_index[0].astype(jnp.int32)
    dst = edge_index[1].astype(jnp.int32)
    dst1 = dst.reshape(NW, NCH1, CHUNK)
    src2 = src * 2
    src3 = jnp.stack([src2, src2 + 1]).reshape(NC, NS, NCH3, CHUNK3)
    dst3 = dst.reshape(NS, NCH3, CHUNK3)

    deg_part = _deg_kernel(dst1)
    degT = deg_part[:, :N_NODES].T
    x2 = _scale_call(degT, x)
    x2v = x2.reshape(2 * N_NODES, DH)
    agg = _agg_kernel(src3, dst3, x2v)
    return _mlp_call(agg, x, degT, W_gcn, b_gcn, W1, b1, W2, b2)
